# baseline (speedup 1.0000x reference)
import functools

import jax
import jax.numpy as jnp
from jax import lax
from jax.experimental import pallas as pl
from jax.experimental.pallas import tpu as pltpu
from jax.experimental.pallas import tpu_sc as plsc

VOCAB = 100000
HIDDEN = 768
MAX_POS = 2048
BATCH = 4
SEQ = 2048
EPS = 1e-12

NC = 2
NS = 16
NW = NC * NS
POS_PER_W = SEQ // NW
NV = HIDDEN // 16
INV_H = 1.0 / HIDDEN

_mesh = plsc.VectorSubcoreMesh(core_axis_name="c", subcore_axis_name="s")


@functools.partial(
    pl.kernel,
    mesh=_mesh,
    out_type=jax.ShapeDtypeStruct((BATCH, SEQ, HIDDEN), jnp.float32),
    compiler_params=pltpu.CompilerParams(needs_layout_passes=False),
    scratch_types=[
        pltpu.VMEM((BATCH, POS_PER_W), jnp.int32),
        pltpu.VMEM((POS_PER_W, HIDDEN), jnp.float32),
        pltpu.VMEM((POS_PER_W, HIDDEN), jnp.float32),
        pltpu.VMEM((HIDDEN,), jnp.float32),
        pltpu.VMEM((HIDDEN,), jnp.float32),
        pltpu.SemaphoreType.DMA,
    ],
)
def _emb_ln_kernel(ids_hbm, tok_hbm, pos_hbm, g_hbm, bt_hbm, out_hbm,
                   idx_v, pos_v, tok_v, g_v, b_v, sem):
    wid = lax.axis_index("s") * NC + lax.axis_index("c")
    pbase = wid * POS_PER_W

    pltpu.sync_copy(g_hbm, g_v)
    pltpu.sync_copy(bt_hbm, b_v)
    pltpu.sync_copy(pos_hbm.at[pl.ds(pbase, POS_PER_W)], pos_v)
    for b in range(BATCH):
        pltpu.sync_copy(ids_hbm.at[b, pl.ds(pbase, POS_PER_W)], idx_v.at[b])

    lanes = lax.iota(jnp.int32, 16)

    def lane_allsum(x):
        for k in (8, 4, 2, 1):
            x = x + x.at[lanes ^ k].get(mode="promise_in_bounds")
        return x

    NA = 8

    def row_body(r):
        accs = [jnp.zeros((16,), jnp.float32) for _ in range(NA)]
        acc2s = [jnp.zeros((16,), jnp.float32) for _ in range(NA)]
        for j in range(NV):
            sl = pl.ds(j * 16, 16)
            e = tok_v[r, sl] + pos_v[r, sl]
            tok_v[r, sl] = e
            accs[j % NA] = accs[j % NA] + e
            acc2s[j % NA] = acc2s[j % NA] + e * e
        while len(accs) > 1:
            accs = [a + b for a, b in zip(accs[::2], accs[1::2])]
            acc2s = [a + b for a, b in zip(acc2s[::2], acc2s[1::2])]
        meanv = lane_allsum(accs[0]) * INV_H
        var = lane_allsum(acc2s[0]) * INV_H - meanv * meanv
        xv = var + EPS
        iv = plsc.bitcast(xv, jnp.int32)
        seed = jnp.full((16,), 0x5F3759DF, jnp.int32) - (iv >> 1)
        y = plsc.bitcast(seed, jnp.float32)
        for _ in range(2):
            y = y * (1.5 - 0.5 * xv * y * y)
        ym = meanv * y
        for j in range(NV):
            sl = pl.ds(j * 16, 16)
            e = tok_v[r, sl]
            tok_v[r, sl] = (e * y - ym) * g_v[sl] + b_v[sl]
        return None

    for b in range(BATCH):
        pltpu.async_copy(tok_hbm.at[idx_v.at[b]], tok_v, sem).wait()
        plsc.parallel_loop(0, POS_PER_W, unroll=2)(row_body)
        pltpu.sync_copy(tok_v, out_hbm.at[b, pl.ds(pbase, POS_PER_W)])


def kernel(input_ids, token_table, pos_table, ln_gamma, ln_beta):
    ids = input_ids.astype(jnp.int32)
    return _emb_ln_kernel(ids, token_table, pos_table, ln_gamma, ln_beta)


# speedup vs baseline: 1.2929x; 1.2929x over previous
import functools

import jax
import jax.numpy as jnp
from jax import lax
from jax.experimental import pallas as pl
from jax.experimental.pallas import tpu as pltpu
from jax.experimental.pallas import tpu_sc as plsc

VOCAB = 100000
HIDDEN = 768
MAX_POS = 2048
BATCH = 4
SEQ = 2048
EPS = 1e-12

NC = 2
NS = 16
NW = NC * NS
POS_PER_W = SEQ // NW
NV = HIDDEN // 16
INV_H = 1.0 / HIDDEN

CH = 16
NCHUNK = BATCH * POS_PER_W // CH
SUB = POS_PER_W // CH

_mesh = plsc.VectorSubcoreMesh(core_axis_name="c", subcore_axis_name="s")


@functools.partial(
    pl.kernel,
    mesh=_mesh,
    out_type=jax.ShapeDtypeStruct((BATCH, SEQ, HIDDEN), jnp.float32),
    compiler_params=pltpu.CompilerParams(needs_layout_passes=False),
    scratch_types=[
        pltpu.VMEM((BATCH, POS_PER_W), jnp.int32),
        pltpu.VMEM((POS_PER_W, HIDDEN), jnp.float32),
        pltpu.VMEM((CH, HIDDEN), jnp.float32),
        pltpu.VMEM((CH, HIDDEN), jnp.float32),
        pltpu.VMEM((CH, HIDDEN), jnp.float32),
        pltpu.VMEM((CH, HIDDEN), jnp.float32),
        pltpu.VMEM((HIDDEN,), jnp.float32),
        pltpu.VMEM((HIDDEN,), jnp.float32),
        pltpu.SemaphoreType.DMA,
        pltpu.SemaphoreType.DMA,
        pltpu.SemaphoreType.DMA,
        pltpu.SemaphoreType.DMA,
    ],
)
def _emb_ln_kernel(ids_hbm, tok_hbm, pos_hbm, g_hbm, bt_hbm, out_hbm,
                   idx_v, pos_v, gb0, gb1, ob0, ob1, g_v, b_v,
                   sg0, sg1, so0, so1):
    wid = lax.axis_index("s") * NC + lax.axis_index("c")
    pbase = wid * POS_PER_W

    gbufs = (gb0, gb1)
    obufs = (ob0, ob1)
    gsems = (sg0, sg1)
    osems = (so0, so1)

    pltpu.sync_copy(g_hbm, g_v)
    pltpu.sync_copy(bt_hbm, b_v)
    pltpu.sync_copy(pos_hbm.at[pl.ds(pbase, POS_PER_W)], pos_v)
    for b in range(BATCH):
        pltpu.sync_copy(ids_hbm.at[b, pl.ds(pbase, POS_PER_W)], idx_v.at[b])

    def start_gather(h, s):
        bh = h // SUB
        off = (h % SUB) * CH
        pltpu.make_async_copy(
            tok_hbm.at[idx_v.at[bh, pl.ds(off, CH)]], gbufs[s], gsems[s]
        ).start()

    def wait_gather(s):
        pltpu.make_async_copy(tok_hbm.at[pl.ds(0, CH)], gbufs[s], gsems[s]).wait()

    def start_out(h, s):
        bh = h // SUB
        off = (h % SUB) * CH
        pltpu.make_async_copy(
            obufs[s], out_hbm.at[bh, pl.ds(pbase + off, CH)], osems[s]
        ).start()

    def wait_out(s):
        pltpu.make_async_copy(
            obufs[s], out_hbm.at[0, pl.ds(0, CH)], osems[s]
        ).wait()

    lanes = lax.iota(jnp.int32, 16)

    def lane_allsum(x):
        for k in (8, 4, 2, 1):
            x = x + x.at[lanes ^ k].get(mode="promise_in_bounds")
        return x

    NA = 8

    def compute_chunk(h, s):
        gb = gbufs[s]
        ob = obufs[s]
        off = (h % SUB) * CH

        def row_body(r):
            po = off + r
            accs = [jnp.zeros((16,), jnp.float32) for _ in range(NA)]
            acc2s = [jnp.zeros((16,), jnp.float32) for _ in range(NA)]
            es = []
            for j in range(NV):
                sl = pl.ds(j * 16, 16)
                e = gb[r, sl] + pos_v[po, sl]
                es.append(e)
                accs[j % NA] = accs[j % NA] + e
                acc2s[j % NA] = acc2s[j % NA] + e * e
            while len(accs) > 1:
                accs = [a + b for a, b in zip(accs[::2], accs[1::2])]
                acc2s = [a + b for a, b in zip(acc2s[::2], acc2s[1::2])]
            meanv = lane_allsum(accs[0]) * INV_H
            var = lane_allsum(acc2s[0]) * INV_H - meanv * meanv
            xv = var + EPS
            iv = plsc.bitcast(xv, jnp.int32)
            seed = jnp.full((16,), 0x5F3759DF, jnp.int32) - (iv >> 1)
            y = plsc.bitcast(seed, jnp.float32)
            for _ in range(2):
                y = y * (1.5 - 0.5 * xv * y * y)
            ym = meanv * y
            for j in range(NV):
                sl = pl.ds(j * 16, 16)
                ob[r, sl] = (es[j] * y - ym) * g_v[sl] + b_v[sl]
            return None

        plsc.parallel_loop(0, CH, unroll=2)(row_body)

    start_gather(0, 0)
    start_gather(1, 1)

    def pair_body(i, carry):
        for s in range(2):
            h = 2 * i + s

            @pl.when(i > 0)
            def _():
                wait_out(s)

            wait_gather(s)
            compute_chunk(h, s)

            @pl.when(h + 2 < NCHUNK)
            def _():
                start_gather(h + 2, s)

            start_out(h, s)
        return carry

    lax.fori_loop(0, NCHUNK // 2, pair_body, 0)
    wait_out(0)
    wait_out(1)


def kernel(input_ids, token_table, pos_table, ln_gamma, ln_beta):
    ids = input_ids.astype(jnp.int32)
    return _emb_ln_kernel(ids, token_table, pos_table, ln_gamma, ln_beta)
